# baseline (speedup 1.0000x reference)
import jax
import jax.numpy as jnp
from jax.experimental import pallas as pl
from jax.experimental.pallas import tpu as pltpu

N_E = 1024
E_DIM = 256
N_B = 16
TOK = 1024
BETA = 0.25
N_TOTAL = N_B * TOK


def _vq_kernel(z_ref, z2_ref, w_ref, loss_ref, zq_ref, ppl_ref, enc_ref,
               idx_ref, counts_ref, loss_acc_ref, w2_ref, eye_ref):
    b = pl.program_id(0)
    zf = z_ref[...]
    w = w_ref[...]

    @pl.when(b == 0)
    def _init():
        ri = jax.lax.broadcasted_iota(jnp.int32, (TOK, TOK), 0)
        ci = jax.lax.broadcasted_iota(jnp.int32, (TOK, TOK), 1)
        eye_ref[...] = jnp.where(ri == ci, 1.0, 0.0).astype(jnp.float32)
        w2_col = jnp.sum(w * w, axis=1, keepdims=True)
        w2_ref[...] = jax.lax.dot_general(
            w2_col, eye_ref[...], (((0,), (0,)), ((), ())),
            preferred_element_type=jnp.float32,
            precision=jax.lax.Precision.HIGHEST)
        counts_ref[...] = jnp.zeros_like(counts_ref)
        loss_acc_ref[...] = jnp.zeros_like(loss_acc_ref)

    z2_col = jax.lax.dot_general(
        eye_ref[...], z2_ref[0], (((1,), (1,)), ((), ())),
        preferred_element_type=jnp.float32,
        precision=jax.lax.Precision.HIGHEST)

    s = jax.lax.dot_general(zf.astype(jnp.bfloat16), w.astype(jnp.bfloat16),
                            (((1,), (1,)), ((), ())),
                            preferred_element_type=jnp.float32)
    d = (z2_col + w2_ref[...]) - 2.0 * s

    mins = jnp.min(d, axis=1, keepdims=True)
    cidx = jax.lax.broadcasted_iota(jnp.int32, (TOK, N_E), 1)
    idx_col = jnp.min(jnp.where(d == mins, cidx, N_E), axis=1,
                      keepdims=True)
    idx_ref[0] = jax.lax.dot_general(
        idx_col.astype(jnp.float32), eye_ref[...], (((0,), (0,)), ((), ())),
        preferred_element_type=jnp.float32,
        precision=jax.lax.Precision.HIGHEST).astype(jnp.int32)

    code_iota = jax.lax.broadcasted_iota(jnp.int32, (TOK, N_E), 1)
    hit = idx_col == code_iota
    enc_ref[...] = jnp.where(hit, 1.0, 0.0).astype(jnp.float32)
    onehot_bf = jnp.where(hit, 1.0, 0.0).astype(jnp.bfloat16)

    zq = jax.lax.dot_general(onehot_bf, w.astype(jnp.bfloat16),
                             (((1,), (0,)), ((), ())),
                             preferred_element_type=jnp.float32)
    r = zq - zf
    zq_ref[...] = zf + r

    ones_row = jnp.full((1, TOK), 1.0, dtype=jnp.bfloat16)
    counts_ref[...] += jax.lax.dot_general(
        ones_row, onehot_bf, (((1,), (0,)), ((), ())),
        preferred_element_type=jnp.float32)
    loss_acc_ref[...] += jnp.sum(r * r, axis=0, keepdims=True)

    @pl.when(b == N_B - 1)
    def _finalize():
        m = jnp.sum(loss_acc_ref[...]) / jnp.float32(N_TOTAL * E_DIM)
        loss_ref[...] = (m + BETA * m).reshape(1, 1)
        e_mean = counts_ref[...] * (1.0 / N_TOTAL)
        ppl_ref[...] = jnp.exp(
            -jnp.sum(e_mean * jnp.log(e_mean + 1e-10))).reshape(1, 1)


@jax.jit
def kernel(z, W):
    zf = jnp.transpose(z, (0, 2, 3, 1)).reshape(N_TOTAL, E_DIM)
    z2 = jnp.sum(z * z, axis=1).reshape(N_B, 1, TOK)
    loss, zq, ppl, enc, idxs = pl.pallas_call(
        _vq_kernel,
        grid=(N_B,),
        in_specs=[
            pl.BlockSpec((TOK, E_DIM), lambda b: (b, 0)),
            pl.BlockSpec((1, 1, TOK), lambda b: (b, 0, 0)),
            pl.BlockSpec((N_E, E_DIM), lambda b: (0, 0)),
        ],
        out_specs=[
            pl.BlockSpec((1, 1), lambda b: (0, 0)),
            pl.BlockSpec((TOK, E_DIM), lambda b: (b, 0)),
            pl.BlockSpec((1, 1), lambda b: (0, 0)),
            pl.BlockSpec((TOK, N_E), lambda b: (b, 0)),
            pl.BlockSpec((1, 1, TOK), lambda b: (b, 0, 0)),
        ],
        out_shape=[
            jax.ShapeDtypeStruct((1, 1), jnp.float32),
            jax.ShapeDtypeStruct((N_TOTAL, E_DIM), jnp.float32),
            jax.ShapeDtypeStruct((1, 1), jnp.float32),
            jax.ShapeDtypeStruct((N_TOTAL, N_E), jnp.float32),
            jax.ShapeDtypeStruct((N_B, 1, TOK), jnp.int32),
        ],
        scratch_shapes=[
            pltpu.VMEM((1, N_E), jnp.float32),
            pltpu.VMEM((1, E_DIM), jnp.float32),
            pltpu.VMEM((1, N_E), jnp.float32),
            pltpu.VMEM((TOK, TOK), jnp.float32),
        ],
    )(zf, z2, W)
    z_q = jnp.transpose(zq.reshape(N_B, 32, 32, E_DIM), (0, 3, 1, 2))
    return (loss.reshape(()),
            z_q,
            ppl.reshape(()),
            enc,
            idxs.reshape(N_TOTAL, 1))


# speedup vs baseline: 1.4464x; 1.4464x over previous
import jax
import jax.numpy as jnp
from jax.experimental import pallas as pl
from jax.experimental.pallas import tpu as pltpu

N_E = 1024
E_DIM = 256
N_B = 16
TOK = 1024
BETA = 0.25
N_TOTAL = N_B * TOK


def _vq_kernel(z_ref, z2_ref, w_ref, w2_ref, loss_ref, zq_ref, ppl_ref,
               enc_ref, idx_ref, counts_ref, loss_acc_ref):
    b = pl.program_id(0)
    zf = z_ref[...]
    w = w_ref[...]

    @pl.when(b == 0)
    def _init():
        counts_ref[...] = jnp.zeros_like(counts_ref)
        loss_acc_ref[...] = jnp.zeros_like(loss_acc_ref)

    z2_col = jnp.transpose(z2_ref[0], (1, 0))

    s = jax.lax.dot_general(zf.astype(jnp.bfloat16), w.astype(jnp.bfloat16),
                            (((1,), (1,)), ((), ())),
                            preferred_element_type=jnp.float32)
    d = (z2_col + w2_ref[...]) - 2.0 * s

    mins = jnp.min(d, axis=1, keepdims=True)
    cidx = jax.lax.broadcasted_iota(jnp.int32, (TOK, N_E), 1)
    idx_col = jnp.min(jnp.where(d == mins, cidx, N_E), axis=1,
                      keepdims=True)
    idx_ref[0] = jnp.transpose(idx_col, (1, 0))

    hit = idx_col == cidx
    enc_ref[...] = jnp.where(hit, 1.0, 0.0).astype(jnp.float32)
    onehot_bf = jnp.where(hit, 1.0, 0.0).astype(jnp.bfloat16)

    zq = jax.lax.dot_general(onehot_bf, w.astype(jnp.bfloat16),
                             (((1,), (0,)), ((), ())),
                             preferred_element_type=jnp.float32)
    r = zq - zf
    zq_ref[...] = zf + r

    ones_row = jnp.full((1, TOK), 1.0, dtype=jnp.bfloat16)
    counts_ref[...] += jax.lax.dot_general(
        ones_row, onehot_bf, (((1,), (0,)), ((), ())),
        preferred_element_type=jnp.float32)
    loss_acc_ref[...] += jnp.sum(r * r, axis=0, keepdims=True)

    @pl.when(b == N_B - 1)
    def _finalize():
        m = jnp.sum(loss_acc_ref[...]) / jnp.float32(N_TOTAL * E_DIM)
        loss_ref[...] = (m + BETA * m).reshape(1, 1)
        e_mean = counts_ref[...] * (1.0 / N_TOTAL)
        ppl_ref[...] = jnp.exp(
            -jnp.sum(e_mean * jnp.log(e_mean + 1e-10))).reshape(1, 1)


@jax.jit
def kernel(z, W):
    zf = jnp.transpose(z, (0, 2, 3, 1)).reshape(N_TOTAL, E_DIM)
    z2 = jnp.sum(z * z, axis=1).reshape(N_B, 1, TOK)
    w2 = jnp.sum(W * W, axis=1).reshape(1, N_E)
    loss, zq, ppl, enc, idxs = pl.pallas_call(
        _vq_kernel,
        grid=(N_B,),
        in_specs=[
            pl.BlockSpec((TOK, E_DIM), lambda b: (b, 0)),
            pl.BlockSpec((1, 1, TOK), lambda b: (b, 0, 0)),
            pl.BlockSpec((N_E, E_DIM), lambda b: (0, 0)),
            pl.BlockSpec((1, N_E), lambda b: (0, 0)),
        ],
        out_specs=[
            pl.BlockSpec((1, 1), lambda b: (0, 0)),
            pl.BlockSpec((TOK, E_DIM), lambda b: (b, 0)),
            pl.BlockSpec((1, 1), lambda b: (0, 0)),
            pl.BlockSpec((TOK, N_E), lambda b: (b, 0)),
            pl.BlockSpec((1, 1, TOK), lambda b: (b, 0, 0)),
        ],
        out_shape=[
            jax.ShapeDtypeStruct((1, 1), jnp.float32),
            jax.ShapeDtypeStruct((N_TOTAL, E_DIM), jnp.float32),
            jax.ShapeDtypeStruct((1, 1), jnp.float32),
            jax.ShapeDtypeStruct((N_TOTAL, N_E), jnp.float32),
            jax.ShapeDtypeStruct((N_B, 1, TOK), jnp.int32),
        ],
        scratch_shapes=[
            pltpu.VMEM((1, N_E), jnp.float32),
            pltpu.VMEM((1, E_DIM), jnp.float32),
        ],
    )(zf, z2, W, w2)
    z_q = jnp.transpose(zq.reshape(N_B, 32, 32, E_DIM), (0, 3, 1, 2))
    return (loss.reshape(()),
            z_q,
            ppl.reshape(()),
            enc,
            idxs.reshape(N_TOTAL, 1))


# speedup vs baseline: 1.4616x; 1.0105x over previous
import jax
import jax.numpy as jnp
from jax.experimental import pallas as pl
from jax.experimental.pallas import tpu as pltpu

N_E = 1024
E_DIM = 256
N_B = 16
TOK = 1024
BETA = 0.25
N_TOTAL = N_B * TOK


def _vq_kernel(z_ref, z2_ref, w_ref, w2_ref, loss_ref, zq_ref, ppl_ref,
               enc_ref, idx_ref, counts_ref, loss_acc_ref):
    b = pl.program_id(0)
    zf = z_ref[...]
    w_bf = w_ref[...]

    @pl.when(b == 0)
    def _init():
        counts_ref[...] = jnp.zeros_like(counts_ref)
        loss_acc_ref[...] = jnp.zeros_like(loss_acc_ref)

    z2_col = jnp.transpose(z2_ref[0], (1, 0))

    s = jax.lax.dot_general(zf.astype(jnp.bfloat16), w_bf,
                            (((1,), (1,)), ((), ())),
                            preferred_element_type=jnp.float32)
    d = (z2_col + w2_ref[...]) - 2.0 * s

    mins = jnp.min(d, axis=1, keepdims=True)
    cidx = jax.lax.broadcasted_iota(jnp.int32, (TOK, N_E), 1)
    idx_col = jnp.min(jnp.where(d == mins, cidx, N_E), axis=1,
                      keepdims=True)
    idx_ref[0] = jnp.transpose(idx_col, (1, 0))

    hit = idx_col == cidx
    onehot = jnp.where(hit, 1.0, 0.0).astype(jnp.float32)
    enc_ref[...] = onehot
    onehot_bf = onehot.astype(jnp.bfloat16)

    zq = jax.lax.dot_general(onehot_bf, w_bf,
                             (((1,), (0,)), ((), ())),
                             preferred_element_type=jnp.float32)
    r = zq - zf
    zq_ref[...] = zf + r

    ones_row = jnp.full((1, TOK), 1.0, dtype=jnp.bfloat16)
    counts_ref[...] += jax.lax.dot_general(
        ones_row, onehot_bf, (((1,), (0,)), ((), ())),
        preferred_element_type=jnp.float32)
    loss_acc_ref[...] += jnp.sum(r * r, axis=0, keepdims=True)

    @pl.when(b == N_B - 1)
    def _finalize():
        m = jnp.sum(loss_acc_ref[...]) / jnp.float32(N_TOTAL * E_DIM)
        loss_ref[...] = (m + BETA * m).reshape(1, 1)
        e_mean = counts_ref[...] * (1.0 / N_TOTAL)
        ppl_ref[...] = jnp.exp(
            -jnp.sum(e_mean * jnp.log(e_mean + 1e-10))).reshape(1, 1)


@jax.jit
def kernel(z, W):
    zf = jnp.transpose(z, (0, 2, 3, 1)).reshape(N_TOTAL, E_DIM)
    z2 = jnp.sum(z * z, axis=1).reshape(N_B, 1, TOK)
    w2 = jnp.sum(W * W, axis=1).reshape(1, N_E)
    w_bf = W.astype(jnp.bfloat16)
    loss, zq, ppl, enc, idxs = pl.pallas_call(
        _vq_kernel,
        grid=(N_B,),
        in_specs=[
            pl.BlockSpec((TOK, E_DIM), lambda b: (b, 0)),
            pl.BlockSpec((1, 1, TOK), lambda b: (b, 0, 0)),
            pl.BlockSpec((N_E, E_DIM), lambda b: (0, 0)),
            pl.BlockSpec((1, N_E), lambda b: (0, 0)),
        ],
        out_specs=[
            pl.BlockSpec((1, 1), lambda b: (0, 0)),
            pl.BlockSpec((TOK, E_DIM), lambda b: (b, 0)),
            pl.BlockSpec((1, 1), lambda b: (0, 0)),
            pl.BlockSpec((TOK, N_E), lambda b: (b, 0)),
            pl.BlockSpec((1, 1, TOK), lambda b: (b, 0, 0)),
        ],
        out_shape=[
            jax.ShapeDtypeStruct((1, 1), jnp.float32),
            jax.ShapeDtypeStruct((N_TOTAL, E_DIM), jnp.float32),
            jax.ShapeDtypeStruct((1, 1), jnp.float32),
            jax.ShapeDtypeStruct((N_TOTAL, N_E), jnp.float32),
            jax.ShapeDtypeStruct((N_B, 1, TOK), jnp.int32),
        ],
        scratch_shapes=[
            pltpu.VMEM((1, N_E), jnp.float32),
            pltpu.VMEM((1, E_DIM), jnp.float32),
        ],
    )(zf, z2, w_bf, w2)
    z_q = jnp.transpose(zq.reshape(N_B, 32, 32, E_DIM), (0, 3, 1, 2))
    return (loss.reshape(()),
            z_q,
            ppl.reshape(()),
            enc,
            idxs.reshape(N_TOTAL, 1))


# speedup vs baseline: 1.4650x; 1.0024x over previous
import jax
import jax.numpy as jnp
from jax.experimental import pallas as pl
from jax.experimental.pallas import tpu as pltpu

N_E = 1024
E_DIM = 256
N_B = 16
N_BLK = 8
TOK = 2048
BETA = 0.25
N_TOTAL = N_BLK * TOK


def _vq_kernel(z_ref, z2_ref, w_ref, w2_ref, loss_ref, zq_ref, ppl_ref,
               enc_ref, idx_ref, counts_ref, loss_acc_ref):
    b = pl.program_id(0)
    zf = z_ref[...]
    w_bf = w_ref[...]

    @pl.when(b == 0)
    def _init():
        counts_ref[...] = jnp.zeros_like(counts_ref)
        loss_acc_ref[...] = jnp.zeros_like(loss_acc_ref)

    z2_col = jnp.transpose(z2_ref[0], (1, 0))

    s = jax.lax.dot_general(zf.astype(jnp.bfloat16), w_bf,
                            (((1,), (1,)), ((), ())),
                            preferred_element_type=jnp.float32)
    d = (z2_col + w2_ref[...]) - 2.0 * s

    mins = jnp.min(d, axis=1, keepdims=True)
    cidx = jax.lax.broadcasted_iota(jnp.int32, (TOK, N_E), 1)
    idx_col = jnp.min(jnp.where(d == mins, cidx, N_E), axis=1,
                      keepdims=True)
    idx_ref[0] = jnp.transpose(idx_col, (1, 0))

    hit = idx_col == cidx
    onehot = jnp.where(hit, 1.0, 0.0).astype(jnp.float32)
    enc_ref[...] = onehot
    onehot_bf = onehot.astype(jnp.bfloat16)

    zq = jax.lax.dot_general(onehot_bf, w_bf,
                             (((1,), (0,)), ((), ())),
                             preferred_element_type=jnp.float32)
    r = zq - zf
    zq_ref[...] = zf + r

    ones_row = jnp.full((1, TOK), 1.0, dtype=jnp.bfloat16)
    counts_ref[...] += jax.lax.dot_general(
        ones_row, onehot_bf, (((1,), (0,)), ((), ())),
        preferred_element_type=jnp.float32)
    loss_acc_ref[...] += jnp.sum(r * r, axis=0, keepdims=True)

    @pl.when(b == N_BLK - 1)
    def _finalize():
        m = jnp.sum(loss_acc_ref[...]) / jnp.float32(N_TOTAL * E_DIM)
        loss_ref[...] = (m + BETA * m).reshape(1, 1)
        e_mean = counts_ref[...] * (1.0 / N_TOTAL)
        ppl_ref[...] = jnp.exp(
            -jnp.sum(e_mean * jnp.log(e_mean + 1e-10))).reshape(1, 1)


@jax.jit
def kernel(z, W):
    zf = jnp.transpose(z, (0, 2, 3, 1)).reshape(N_TOTAL, E_DIM)
    z2 = jnp.sum(z * z, axis=1).reshape(N_BLK, 1, TOK)
    w2 = jnp.sum(W * W, axis=1).reshape(1, N_E)
    w_bf = W.astype(jnp.bfloat16)
    loss, zq, ppl, enc, idxs = pl.pallas_call(
        _vq_kernel,
        grid=(N_BLK,),
        in_specs=[
            pl.BlockSpec((TOK, E_DIM), lambda b: (b, 0)),
            pl.BlockSpec((1, 1, TOK), lambda b: (b, 0, 0)),
            pl.BlockSpec((N_E, E_DIM), lambda b: (0, 0)),
            pl.BlockSpec((1, N_E), lambda b: (0, 0)),
        ],
        out_specs=[
            pl.BlockSpec((1, 1), lambda b: (0, 0)),
            pl.BlockSpec((TOK, E_DIM), lambda b: (b, 0)),
            pl.BlockSpec((1, 1), lambda b: (0, 0)),
            pl.BlockSpec((TOK, N_E), lambda b: (b, 0)),
            pl.BlockSpec((1, 1, TOK), lambda b: (b, 0, 0)),
        ],
        out_shape=[
            jax.ShapeDtypeStruct((1, 1), jnp.float32),
            jax.ShapeDtypeStruct((N_TOTAL, E_DIM), jnp.float32),
            jax.ShapeDtypeStruct((1, 1), jnp.float32),
            jax.ShapeDtypeStruct((N_TOTAL, N_E), jnp.float32),
            jax.ShapeDtypeStruct((N_BLK, 1, TOK), jnp.int32),
        ],
        scratch_shapes=[
            pltpu.VMEM((1, N_E), jnp.float32),
            pltpu.VMEM((1, E_DIM), jnp.float32),
        ],
    )(zf, z2, w_bf, w2)
    z_q = jnp.transpose(zq.reshape(N_B, 32, 32, E_DIM), (0, 3, 1, 2))
    return (loss.reshape(()),
            z_q,
            ppl.reshape(()),
            enc,
            idxs.reshape(N_TOTAL, 1))
